# baseline (speedup 1.0000x reference)
import functools

import jax
import jax.numpy as jnp
from jax import lax
from jax.experimental import pallas as pl
from jax.experimental.pallas import tpu as pltpu
from jax.experimental.pallas import tpu_sc as plsc

N = 10000
E = 320000
F = 128
HID = 128
C = 40

NC = 2
NS = 16
NW = NC * NS
EPW = E // NW
CH = 128
NCH = 80
KB = 4
NBLK = NCH // KB
TRASH = N
NH = 10240
HPT = NH // NS
NP = 10240
RPT = NP // NS

_mesh = plsc.VectorSubcoreMesh(core_axis_name="c", subcore_axis_name="s")


@functools.partial(
    pl.kernel,
    out_type=jax.ShapeDtypeStruct((NC, 2, NH), jnp.float32),
    mesh=_mesh,
    scratch_types=[
        pltpu.VMEM((2, NCH, CH), jnp.int32),
        pltpu.VMEM((CH,), jnp.float32),
        pltpu.VMEM((HPT,), jnp.float32),
        pltpu.VMEM_SHARED((NH,), jnp.float32),
        pltpu.VMEM_SHARED((NH,), jnp.float32),
    ],
)
def _degree_kernel(edge_hbm, out_hbm, idx_v, ones_v, zeros_v, hsrc_s, hdst_s):
    c = lax.axis_index("c")
    s = lax.axis_index("s")
    wid = s * NC + c
    pltpu.sync_copy(edge_hbm.at[wid], idx_v)
    one = jnp.ones((16,), jnp.float32)
    zero = jnp.zeros((16,), jnp.float32)
    for q in range(CH // 16):
        ones_v[pl.ds(q * 16, 16)] = one
    for q in range(HPT // 16):
        zeros_v[pl.ds(q * 16, 16)] = zero
    sl = pl.ds(s * HPT, HPT)
    pltpu.sync_copy(zeros_v, hsrc_s.at[sl])
    pltpu.sync_copy(zeros_v, hdst_s.at[sl])
    plsc.subcore_barrier()

    def body(i, carry):
        pltpu.sync_copy(ones_v, hsrc_s.at[idx_v.at[0, i]], add=True)
        pltpu.sync_copy(ones_v, hdst_s.at[idx_v.at[1, i]], add=True)
        return carry

    lax.fori_loop(0, NCH, body, 0)
    plsc.subcore_barrier()
    pltpu.sync_copy(hsrc_s.at[sl], out_hbm.at[c, 0, sl])
    pltpu.sync_copy(hdst_s.at[sl], out_hbm.at[c, 1, sl])


@functools.partial(
    pl.kernel,
    out_type=jax.ShapeDtypeStruct((NC, NP, HID), jnp.float32),
    mesh=_mesh,
    scratch_types=[
        pltpu.VMEM((2, 2, KB, CH), jnp.int32),
        pltpu.VMEM((2, CH, HID), jnp.float32),
        pltpu.VMEM_SHARED((NP, HID), jnp.float32),
        pltpu.SemaphoreType.DMA,
        pltpu.SemaphoreType.DMA,
        pltpu.SemaphoreType.DMA,
        pltpu.SemaphoreType.DMA,
    ],
)
def _agg_kernel(hn_hbm, edge_hbm, out_hbm, idx_v, rows_v, acc_s,
                gsem0, gsem1, ssem0, ssem1):
    c = lax.axis_index("c")
    s = lax.axis_index("s")
    wid = s * NC + c
    gsems = (gsem0, gsem1)
    ssems = (ssem0, ssem1)

    zero = jnp.zeros((16,), jnp.float32)
    for r in range(CH):
        for q in range(HID // 16):
            rows_v[0, r, pl.ds(q * 16, 16)] = zero
    for k in range(RPT // CH):
        pltpu.sync_copy(rows_v.at[0], acc_s.at[pl.ds(s * RPT + k * CH, CH)])
    plsc.subcore_barrier()

    pltpu.sync_copy(edge_hbm.at[wid, :, pl.ds(0, KB)], idx_v.at[0])
    pltpu.async_copy(hn_hbm.at[idx_v.at[0, 0, 0]], rows_v.at[0], gsems[0])

    def block(k, carry):
        kb = lax.rem(k, 2)
        kb1 = lax.rem(k + 1, 2)
        for m in range(KB):
            b = m % 2
            b1 = (m + 1) % 2
            pltpu.make_async_copy(
                hn_hbm.at[idx_v.at[kb, 0, m]], rows_v.at[b], gsems[b]).wait()
            pltpu.async_copy(rows_v.at[b], acc_s.at[idx_v.at[kb, 1, m]],
                             ssems[b], add=True)
            if m == 0:
                @pl.when(k > 0)
                def _():
                    pltpu.make_async_copy(
                        rows_v.at[b1], acc_s.at[idx_v.at[kb, 1, 0]],
                        ssems[b1]).wait()
            else:
                pltpu.make_async_copy(
                    rows_v.at[b1], acc_s.at[idx_v.at[kb, 1, m - 1]],
                    ssems[b1]).wait()
            if m < KB - 1:
                pltpu.async_copy(
                    hn_hbm.at[idx_v.at[kb, 0, m + 1]], rows_v.at[b1],
                    gsems[b1])
            else:
                @pl.when(k < NBLK - 1)
                def _():
                    pltpu.async_copy(
                        hn_hbm.at[idx_v.at[kb1, 0, 0]], rows_v.at[b1],
                        gsems[b1])
            if m == 0:
                @pl.when(k < NBLK - 1)
                def _():
                    pltpu.sync_copy(
                        edge_hbm.at[wid, :, pl.ds((k + 1) * KB, KB)],
                        idx_v.at[kb1])
        return carry

    lax.fori_loop(0, NBLK, block, 0)
    pltpu.make_async_copy(
        rows_v.at[1], acc_s.at[idx_v.at[1, 1, KB - 1]], ssems[1]).wait()
    plsc.subcore_barrier()
    sl = pl.ds(s * RPT, RPT)
    pltpu.sync_copy(acc_s.at[sl], out_hbm.at[c, sl])


BM = 400
GRID = N // BM


def _rsqrt_deg(h):
    return lax.rsqrt(jnp.maximum(h, 1.0))


def _proj_body(nf_ref, wp_ref, bp_ref, hp_ref, out_ref):
    ns = _rsqrt_deg(hp_ref[0, 0] + hp_ref[1, 0])
    x = jnp.dot(nf_ref[...], wp_ref[...],
                preferred_element_type=jnp.float32) + bp_ref[...]
    out_ref[...] = x * ns


def _conv_body(aggp_ref, hp_ref, w_ref, b_ref, out_ref):
    a = aggp_ref[0] + aggp_ref[1]
    nd = _rsqrt_deg(hp_ref[0, 1] + hp_ref[1, 1])
    ns = _rsqrt_deg(hp_ref[0, 0] + hp_ref[1, 0])
    h = jnp.maximum(
        jnp.dot(a * nd, w_ref[...], preferred_element_type=jnp.float32)
        + b_ref[...], 0.0)
    out_ref[...] = h * ns


def _head_body(aggp_ref, hp_ref, w2_ref, b2_ref, wc_ref, bc_ref, out_ref):
    a = aggp_ref[0] + aggp_ref[1]
    nd = _rsqrt_deg(hp_ref[0, 1] + hp_ref[1, 1])
    h = jnp.maximum(
        jnp.dot(a * nd, w2_ref[...], preferred_element_type=jnp.float32)
        + b2_ref[...], 0.0)
    out_ref[...] = jnp.dot(h, wc_ref[...],
                           preferred_element_type=jnp.float32) + bc_ref[...]


def _hp_spec():
    return pl.BlockSpec((NC, 2, BM, 1), lambda m: (0, 0, m, 0))


def _proj_call(n_feats, W_proj, b_proj2, hp):
    return pl.pallas_call(
        _proj_body,
        grid=(GRID,),
        in_specs=[
            pl.BlockSpec((BM, F), lambda m: (m, 0)),
            pl.BlockSpec((F, HID), lambda m: (0, 0)),
            pl.BlockSpec((1, HID), lambda m: (0, 0)),
            _hp_spec(),
        ],
        out_specs=pl.BlockSpec((BM, HID), lambda m: (m, 0)),
        out_shape=jax.ShapeDtypeStruct((N, HID), jnp.float32),
    )(n_feats, W_proj, b_proj2, hp)


def _conv_call(aggp, hp, W, b2):
    return pl.pallas_call(
        _conv_body,
        grid=(GRID,),
        in_specs=[
            pl.BlockSpec((NC, BM, HID), lambda m: (0, m, 0)),
            _hp_spec(),
            pl.BlockSpec((HID, HID), lambda m: (0, 0)),
            pl.BlockSpec((1, HID), lambda m: (0, 0)),
        ],
        out_specs=pl.BlockSpec((BM, HID), lambda m: (m, 0)),
        out_shape=jax.ShapeDtypeStruct((N, HID), jnp.float32),
    )(aggp, hp, W, b2)


def _head_call(aggp, hp, W2, b22, W_cls, b_cls2):
    return pl.pallas_call(
        _head_body,
        grid=(GRID,),
        in_specs=[
            pl.BlockSpec((NC, BM, HID), lambda m: (0, m, 0)),
            _hp_spec(),
            pl.BlockSpec((HID, HID), lambda m: (0, 0)),
            pl.BlockSpec((1, HID), lambda m: (0, 0)),
            pl.BlockSpec((HID, C), lambda m: (0, 0)),
            pl.BlockSpec((1, C), lambda m: (0, 0)),
        ],
        out_specs=pl.BlockSpec((BM, C), lambda m: (m, 0)),
        out_shape=jax.ShapeDtypeStruct((N, C), jnp.float32),
    )(aggp, hp, W2, b22, W_cls, b_cls2)


def kernel(edge_index, n_feats, W_proj, b_proj, W1, b1, W2, b2, W_cls, b_cls):
    ei = edge_index.reshape(2, NW, EPW)
    npad = NCH * CH - EPW
    pad_deg = jnp.full((2, NW, npad), TRASH, jnp.int32)
    pad_agg = jnp.stack([
        jnp.zeros((NW, npad), jnp.int32),
        jnp.full((NW, npad), TRASH, jnp.int32)])
    edge_deg = jnp.concatenate([ei, pad_deg], axis=2) \
        .reshape(2, NW, NCH, CH).transpose(1, 0, 2, 3)
    edge_agg = jnp.concatenate([ei, pad_agg], axis=2) \
        .reshape(2, NW, NCH, CH).transpose(1, 0, 2, 3)

    hp = _degree_kernel(edge_deg)
    hp4 = hp.reshape(NC, 2, NH, 1)
    hn1 = _proj_call(n_feats, W_proj, b_proj.reshape(1, HID), hp4)
    p1 = _agg_kernel(hn1, edge_agg)
    hn2 = _conv_call(p1, hp4, W1, b1.reshape(1, HID))
    p2 = _agg_kernel(hn2, edge_agg)
    out = _head_call(p2, hp4, W2, b2.reshape(1, HID), W_cls,
                     b_cls.reshape(1, C))
    return out


# speedup vs baseline: 1.0070x; 1.0070x over previous
import functools

import jax
import jax.numpy as jnp
from jax import lax
from jax.experimental import pallas as pl
from jax.experimental.pallas import tpu as pltpu
from jax.experimental.pallas import tpu_sc as plsc

N = 10000
E = 320000
F = 128
HID = 128
C = 40

NC = 2
NS = 16
NW = NC * NS
EPW = E // NW
CH = 64
NCH = 160
KB = 8
NSLOT = 4
NBLK = NCH // KB
TRASH = N
NH = 10240
HPT = NH // NS
NP = 10240
RPT = NP // NS

_mesh = plsc.VectorSubcoreMesh(core_axis_name="c", subcore_axis_name="s")


@functools.partial(
    pl.kernel,
    out_type=jax.ShapeDtypeStruct((NC, 2, NH), jnp.float32),
    mesh=_mesh,
    scratch_types=[
        pltpu.VMEM((2, NCH, CH), jnp.int32),
        pltpu.VMEM((CH,), jnp.float32),
        pltpu.VMEM((HPT,), jnp.float32),
        pltpu.VMEM_SHARED((NH,), jnp.float32),
        pltpu.VMEM_SHARED((NH,), jnp.float32),
    ],
)
def _degree_kernel(edge_hbm, out_hbm, idx_v, ones_v, zeros_v, hsrc_s, hdst_s):
    c = lax.axis_index("c")
    s = lax.axis_index("s")
    wid = s * NC + c
    pltpu.sync_copy(edge_hbm.at[wid], idx_v)
    one = jnp.ones((16,), jnp.float32)
    zero = jnp.zeros((16,), jnp.float32)
    for q in range(CH // 16):
        ones_v[pl.ds(q * 16, 16)] = one
    for q in range(HPT // 16):
        zeros_v[pl.ds(q * 16, 16)] = zero
    sl = pl.ds(s * HPT, HPT)
    pltpu.sync_copy(zeros_v, hsrc_s.at[sl])
    pltpu.sync_copy(zeros_v, hdst_s.at[sl])
    plsc.subcore_barrier()

    def body(i, carry):
        pltpu.sync_copy(ones_v, hsrc_s.at[idx_v.at[0, i]], add=True)
        pltpu.sync_copy(ones_v, hdst_s.at[idx_v.at[1, i]], add=True)
        return carry

    lax.fori_loop(0, NCH, body, 0)
    plsc.subcore_barrier()
    pltpu.sync_copy(hsrc_s.at[sl], out_hbm.at[c, 0, sl])
    pltpu.sync_copy(hdst_s.at[sl], out_hbm.at[c, 1, sl])


@functools.partial(
    pl.kernel,
    out_type=jax.ShapeDtypeStruct((NC, NP, HID), jnp.float32),
    mesh=_mesh,
    scratch_types=[
        pltpu.VMEM((2, 2, KB, CH), jnp.int32),
        pltpu.VMEM((NSLOT, CH, HID), jnp.float32),
        pltpu.VMEM_SHARED((NP, HID), jnp.float32),
        [pltpu.SemaphoreType.DMA] * NSLOT,
        [pltpu.SemaphoreType.DMA] * NSLOT,
    ],
)
def _agg_kernel(hn_hbm, edge_hbm, out_hbm, idx_v, rows_v, acc_s,
                gsems, ssems):
    c = lax.axis_index("c")
    s = lax.axis_index("s")
    wid = s * NC + c

    zero = jnp.zeros((16,), jnp.float32)
    for r in range(CH):
        for q in range(HID // 16):
            rows_v[0, r, pl.ds(q * 16, 16)] = zero
    for k in range(RPT // CH):
        pltpu.sync_copy(rows_v.at[0], acc_s.at[pl.ds(s * RPT + k * CH, CH)])
    plsc.subcore_barrier()

    pltpu.sync_copy(edge_hbm.at[wid, :, pl.ds(0, KB)], idx_v.at[0])
    pltpu.async_copy(hn_hbm.at[idx_v.at[0, 0, 0]], rows_v.at[0], gsems[0])
    pltpu.async_copy(hn_hbm.at[idx_v.at[0, 0, 1]], rows_v.at[1], gsems[1])

    def block(k, carry):
        kb = lax.rem(k, 2)
        kb1 = lax.rem(k + 1, 2)
        for m in range(KB):
            b = m % NSLOT
            b2 = (m + 2) % NSLOT
            pltpu.make_async_copy(
                hn_hbm.at[idx_v.at[kb, 0, m]], rows_v.at[b], gsems[b]).wait()
            pltpu.async_copy(rows_v.at[b], acc_s.at[idx_v.at[kb, 1, m]],
                             ssems[b], add=True)
            if m < 2:
                @pl.when(k > 0)
                def _():
                    pltpu.make_async_copy(
                        rows_v.at[b2], acc_s.at[idx_v.at[kb, 1, m]],
                        ssems[b2]).wait()
            else:
                pltpu.make_async_copy(
                    rows_v.at[b2], acc_s.at[idx_v.at[kb, 1, m - 2]],
                    ssems[b2]).wait()
            if m < KB - 2:
                pltpu.async_copy(
                    hn_hbm.at[idx_v.at[kb, 0, m + 2]], rows_v.at[b2],
                    gsems[b2])
            else:
                @pl.when(k < NBLK - 1)
                def _():
                    pltpu.async_copy(
                        hn_hbm.at[idx_v.at[kb1, 0, m + 2 - KB]],
                        rows_v.at[b2], gsems[b2])
            if m == 2:
                @pl.when(k < NBLK - 1)
                def _():
                    pltpu.sync_copy(
                        edge_hbm.at[wid, :, pl.ds((k + 1) * KB, KB)],
                        idx_v.at[kb1])
        return carry

    lax.fori_loop(0, NBLK, block, 0)
    for m in (KB - 2, KB - 1):
        b = m % NSLOT
        pltpu.make_async_copy(
            rows_v.at[b], acc_s.at[idx_v.at[1, 1, m]], ssems[b]).wait()
    plsc.subcore_barrier()
    sl = pl.ds(s * RPT, RPT)
    pltpu.sync_copy(acc_s.at[sl], out_hbm.at[c, sl])


BM = 400
GRID = N // BM


def _rsqrt_deg(h):
    return lax.rsqrt(jnp.maximum(h, 1.0))


def _proj_body(nf_ref, wp_ref, bp_ref, hp_ref, out_ref):
    ns = _rsqrt_deg(hp_ref[0, 0] + hp_ref[1, 0])
    x = jnp.dot(nf_ref[...], wp_ref[...],
                preferred_element_type=jnp.float32) + bp_ref[...]
    out_ref[...] = x * ns


def _conv_body(aggp_ref, hp_ref, w_ref, b_ref, out_ref):
    a = aggp_ref[0] + aggp_ref[1]
    nd = _rsqrt_deg(hp_ref[0, 1] + hp_ref[1, 1])
    ns = _rsqrt_deg(hp_ref[0, 0] + hp_ref[1, 0])
    h = jnp.maximum(
        jnp.dot(a * nd, w_ref[...], preferred_element_type=jnp.float32)
        + b_ref[...], 0.0)
    out_ref[...] = h * ns


def _head_body(aggp_ref, hp_ref, w2_ref, b2_ref, wc_ref, bc_ref, out_ref):
    a = aggp_ref[0] + aggp_ref[1]
    nd = _rsqrt_deg(hp_ref[0, 1] + hp_ref[1, 1])
    h = jnp.maximum(
        jnp.dot(a * nd, w2_ref[...], preferred_element_type=jnp.float32)
        + b2_ref[...], 0.0)
    out_ref[...] = jnp.dot(h, wc_ref[...],
                           preferred_element_type=jnp.float32) + bc_ref[...]


def _hp_spec():
    return pl.BlockSpec((NC, 2, BM, 1), lambda m: (0, 0, m, 0))


def _proj_call(n_feats, W_proj, b_proj2, hp):
    return pl.pallas_call(
        _proj_body,
        grid=(GRID,),
        in_specs=[
            pl.BlockSpec((BM, F), lambda m: (m, 0)),
            pl.BlockSpec((F, HID), lambda m: (0, 0)),
            pl.BlockSpec((1, HID), lambda m: (0, 0)),
            _hp_spec(),
        ],
        out_specs=pl.BlockSpec((BM, HID), lambda m: (m, 0)),
        out_shape=jax.ShapeDtypeStruct((N, HID), jnp.float32),
    )(n_feats, W_proj, b_proj2, hp)


def _conv_call(aggp, hp, W, b2):
    return pl.pallas_call(
        _conv_body,
        grid=(GRID,),
        in_specs=[
            pl.BlockSpec((NC, BM, HID), lambda m: (0, m, 0)),
            _hp_spec(),
            pl.BlockSpec((HID, HID), lambda m: (0, 0)),
            pl.BlockSpec((1, HID), lambda m: (0, 0)),
        ],
        out_specs=pl.BlockSpec((BM, HID), lambda m: (m, 0)),
        out_shape=jax.ShapeDtypeStruct((N, HID), jnp.float32),
    )(aggp, hp, W, b2)


def _head_call(aggp, hp, W2, b22, W_cls, b_cls2):
    return pl.pallas_call(
        _head_body,
        grid=(GRID,),
        in_specs=[
            pl.BlockSpec((NC, BM, HID), lambda m: (0, m, 0)),
            _hp_spec(),
            pl.BlockSpec((HID, HID), lambda m: (0, 0)),
            pl.BlockSpec((1, HID), lambda m: (0, 0)),
            pl.BlockSpec((HID, C), lambda m: (0, 0)),
            pl.BlockSpec((1, C), lambda m: (0, 0)),
        ],
        out_specs=pl.BlockSpec((BM, C), lambda m: (m, 0)),
        out_shape=jax.ShapeDtypeStruct((N, C), jnp.float32),
    )(aggp, hp, W2, b22, W_cls, b_cls2)


def kernel(edge_index, n_feats, W_proj, b_proj, W1, b1, W2, b2, W_cls, b_cls):
    ei = edge_index.reshape(2, NW, EPW)
    npad = NCH * CH - EPW
    pad_deg = jnp.full((2, NW, npad), TRASH, jnp.int32)
    pad_agg = jnp.stack([
        jnp.zeros((NW, npad), jnp.int32),
        jnp.full((NW, npad), TRASH, jnp.int32)])
    edge_deg = jnp.concatenate([ei, pad_deg], axis=2) \
        .reshape(2, NW, NCH, CH).transpose(1, 0, 2, 3)
    edge_agg = jnp.concatenate([ei, pad_agg], axis=2) \
        .reshape(2, NW, NCH, CH).transpose(1, 0, 2, 3)

    hp = _degree_kernel(edge_deg)
    hp4 = hp.reshape(NC, 2, NH, 1)
    hn1 = _proj_call(n_feats, W_proj, b_proj.reshape(1, HID), hp4)
    p1 = _agg_kernel(hn1, edge_agg)
    hn2 = _conv_call(p1, hp4, W1, b1.reshape(1, HID))
    p2 = _agg_kernel(hn2, edge_agg)
    out = _head_call(p2, hp4, W2, b2.reshape(1, HID), W_cls,
                     b_cls.reshape(1, C))
    return out


# speedup vs baseline: 1.0785x; 1.0710x over previous
import functools

import jax
import jax.numpy as jnp
from jax import lax
from jax.experimental import pallas as pl
from jax.experimental.pallas import tpu as pltpu
from jax.experimental.pallas import tpu_sc as plsc

N = 10000
E = 320000
F = 128
HID = 128
C = 40

NC = 2
NS = 16
NW = NC * NS
EPW = E // NW
CH = 128
NCH = 80
KB = 4
NBLK = NCH // KB
TRASH = N
NH = 10240
HPT = NH // NS
NP = 10240
RPT = NP // NS

_mesh = plsc.VectorSubcoreMesh(core_axis_name="c", subcore_axis_name="s")


@functools.partial(
    pl.kernel,
    out_type=jax.ShapeDtypeStruct((NC, 2, NH), jnp.float32),
    mesh=_mesh,
    scratch_types=[
        pltpu.VMEM((2, NCH, CH), jnp.int32),
        pltpu.VMEM((CH,), jnp.float32),
        pltpu.VMEM((HPT,), jnp.float32),
        pltpu.VMEM_SHARED((NH,), jnp.float32),
        pltpu.VMEM_SHARED((NH,), jnp.float32),
    ],
)
def _degree_kernel(edge_hbm, out_hbm, idx_v, ones_v, zeros_v, hsrc_s, hdst_s):
    c = lax.axis_index("c")
    s = lax.axis_index("s")
    wid = s * NC + c
    pltpu.sync_copy(edge_hbm.at[wid], idx_v)
    one = jnp.ones((16,), jnp.float32)
    zero = jnp.zeros((16,), jnp.float32)
    for q in range(CH // 16):
        ones_v[pl.ds(q * 16, 16)] = one
    for q in range(HPT // 16):
        zeros_v[pl.ds(q * 16, 16)] = zero
    sl = pl.ds(s * HPT, HPT)
    pltpu.sync_copy(zeros_v, hsrc_s.at[sl])
    pltpu.sync_copy(zeros_v, hdst_s.at[sl])
    plsc.subcore_barrier()

    def body(i, carry):
        pltpu.sync_copy(ones_v, hsrc_s.at[idx_v.at[0, i]], add=True)
        pltpu.sync_copy(ones_v, hdst_s.at[idx_v.at[1, i]], add=True)
        return carry

    lax.fori_loop(0, NCH, body, 0)
    plsc.subcore_barrier()
    pltpu.sync_copy(hsrc_s.at[sl], out_hbm.at[c, 0, sl])
    pltpu.sync_copy(hdst_s.at[sl], out_hbm.at[c, 1, sl])


@functools.partial(
    pl.kernel,
    out_type=jax.ShapeDtypeStruct((NC, NP, HID), jnp.float32),
    mesh=_mesh,
    scratch_types=[
        pltpu.VMEM((2, 2, KB, CH), jnp.int32),
        pltpu.VMEM((2, CH, HID), jnp.float32),
        pltpu.VMEM_SHARED((NP, HID), jnp.float32),
        pltpu.SemaphoreType.DMA,
        pltpu.SemaphoreType.DMA,
    ],
)
def _agg_kernel(hn_hbm, edge_hbm, out_hbm, idx_v, rows_v, acc_s, sem0, sem1):
    c = lax.axis_index("c")
    s = lax.axis_index("s")
    wid = s * NC + c
    sems = (sem0, sem1)

    zero = jnp.zeros((16,), jnp.float32)
    for r in range(CH):
        for q in range(HID // 16):
            rows_v[0, r, pl.ds(q * 16, 16)] = zero
    for k in range(RPT // CH):
        pltpu.sync_copy(rows_v.at[0], acc_s.at[pl.ds(s * RPT + k * CH, CH)])
    plsc.subcore_barrier()

    pltpu.sync_copy(edge_hbm.at[wid, :, pl.ds(0, KB)], idx_v.at[0])
    pltpu.sync_copy(edge_hbm.at[wid, :, pl.ds(KB, KB)], idx_v.at[1])
    pltpu.async_copy(hn_hbm.at[idx_v.at[0, 0, 0]], rows_v.at[0], sems[0])
    pltpu.async_copy(hn_hbm.at[idx_v.at[0, 0, 1]], rows_v.at[1], sems[1])

    def block(k, carry):
        kb = lax.rem(k, 2)
        kb1 = lax.rem(k + 1, 2)
        for m in range(KB):
            b = m % 2
            pltpu.make_async_copy(
                hn_hbm.at[idx_v.at[kb, 0, m]], rows_v.at[b], sems[b]).wait()
            pltpu.sync_copy(rows_v.at[b], acc_s.at[idx_v.at[kb, 1, m]],
                            add=True)
            if m < KB - 2:
                pltpu.async_copy(
                    hn_hbm.at[idx_v.at[kb, 0, m + 2]], rows_v.at[b], sems[b])
            else:
                @pl.when(k < NBLK - 1)
                def _():
                    pltpu.async_copy(
                        hn_hbm.at[idx_v.at[kb1, 0, m + 2 - KB]],
                        rows_v.at[b], sems[b])
        @pl.when(k < NBLK - 2)
        def _():
            pltpu.sync_copy(
                edge_hbm.at[wid, :, pl.ds((k + 2) * KB, KB)], idx_v.at[kb])
        return carry

    lax.fori_loop(0, NBLK, block, 0)
    plsc.subcore_barrier()
    sl = pl.ds(s * RPT, RPT)
    pltpu.sync_copy(acc_s.at[sl], out_hbm.at[c, sl])


BM = 2000
GRID = N // BM


def _rsqrt_deg(h):
    return lax.rsqrt(jnp.maximum(h, 1.0))


def _proj_body(nf_ref, wp_ref, bp_ref, hp_ref, out_ref):
    ns = _rsqrt_deg(hp_ref[0, 0] + hp_ref[1, 0])
    x = jnp.dot(nf_ref[...], wp_ref[...],
                preferred_element_type=jnp.float32) + bp_ref[...]
    out_ref[...] = x * ns


def _conv_body(aggp_ref, hp_ref, w_ref, b_ref, out_ref):
    a = aggp_ref[0] + aggp_ref[1]
    nd = _rsqrt_deg(hp_ref[0, 1] + hp_ref[1, 1])
    ns = _rsqrt_deg(hp_ref[0, 0] + hp_ref[1, 0])
    h = jnp.maximum(
        jnp.dot(a * nd, w_ref[...], preferred_element_type=jnp.float32)
        + b_ref[...], 0.0)
    out_ref[...] = h * ns


def _head_body(aggp_ref, hp_ref, w2_ref, b2_ref, wc_ref, bc_ref, out_ref):
    a = aggp_ref[0] + aggp_ref[1]
    nd = _rsqrt_deg(hp_ref[0, 1] + hp_ref[1, 1])
    h = jnp.maximum(
        jnp.dot(a * nd, w2_ref[...], preferred_element_type=jnp.float32)
        + b2_ref[...], 0.0)
    out_ref[...] = jnp.dot(h, wc_ref[...],
                           preferred_element_type=jnp.float32) + bc_ref[...]


def _hp_spec():
    return pl.BlockSpec((NC, 2, BM, 1), lambda m: (0, 0, m, 0))


def _proj_call(n_feats, W_proj, b_proj2, hp):
    return pl.pallas_call(
        _proj_body,
        grid=(GRID,),
        in_specs=[
            pl.BlockSpec((BM, F), lambda m: (m, 0)),
            pl.BlockSpec((F, HID), lambda m: (0, 0)),
            pl.BlockSpec((1, HID), lambda m: (0, 0)),
            _hp_spec(),
        ],
        out_specs=pl.BlockSpec((BM, HID), lambda m: (m, 0)),
        out_shape=jax.ShapeDtypeStruct((N, HID), jnp.float32),
    )(n_feats, W_proj, b_proj2, hp)


def _conv_call(aggp, hp, W, b2):
    return pl.pallas_call(
        _conv_body,
        grid=(GRID,),
        in_specs=[
            pl.BlockSpec((NC, BM, HID), lambda m: (0, m, 0)),
            _hp_spec(),
            pl.BlockSpec((HID, HID), lambda m: (0, 0)),
            pl.BlockSpec((1, HID), lambda m: (0, 0)),
        ],
        out_specs=pl.BlockSpec((BM, HID), lambda m: (m, 0)),
        out_shape=jax.ShapeDtypeStruct((N, HID), jnp.float32),
    )(aggp, hp, W, b2)


def _head_call(aggp, hp, W2, b22, W_cls, b_cls2):
    return pl.pallas_call(
        _head_body,
        grid=(GRID,),
        in_specs=[
            pl.BlockSpec((NC, BM, HID), lambda m: (0, m, 0)),
            _hp_spec(),
            pl.BlockSpec((HID, HID), lambda m: (0, 0)),
            pl.BlockSpec((1, HID), lambda m: (0, 0)),
            pl.BlockSpec((HID, C), lambda m: (0, 0)),
            pl.BlockSpec((1, C), lambda m: (0, 0)),
        ],
        out_specs=pl.BlockSpec((BM, C), lambda m: (m, 0)),
        out_shape=jax.ShapeDtypeStruct((N, C), jnp.float32),
    )(aggp, hp, W2, b22, W_cls, b_cls2)


def kernel(edge_index, n_feats, W_proj, b_proj, W1, b1, W2, b2, W_cls, b_cls):
    ei = edge_index.reshape(2, NW, EPW)
    npad = NCH * CH - EPW
    pad_deg = jnp.full((2, NW, npad), TRASH, jnp.int32)
    pad_agg = jnp.stack([
        jnp.zeros((NW, npad), jnp.int32),
        jnp.full((NW, npad), TRASH, jnp.int32)])
    edge_deg = jnp.concatenate([ei, pad_deg], axis=2) \
        .reshape(2, NW, NCH, CH).transpose(1, 0, 2, 3)
    edge_agg = jnp.concatenate([ei, pad_agg], axis=2) \
        .reshape(2, NW, NCH, CH).transpose(1, 0, 2, 3)

    hp = _degree_kernel(edge_deg)
    hp4 = hp.reshape(NC, 2, NH, 1)
    hn1 = _proj_call(n_feats, W_proj, b_proj.reshape(1, HID), hp4)
    p1 = _agg_kernel(hn1, edge_agg)
    hn2 = _conv_call(p1, hp4, W1, b1.reshape(1, HID))
    p2 = _agg_kernel(hn2, edge_agg)
    out = _head_call(p2, hp4, W2, b2.reshape(1, HID), W_cls,
                     b_cls.reshape(1, C))
    return out


# speedup vs baseline: 1.0798x; 1.0012x over previous
import functools

import jax
import jax.numpy as jnp
from jax import lax
from jax.experimental import pallas as pl
from jax.experimental.pallas import tpu as pltpu
from jax.experimental.pallas import tpu_sc as plsc

N = 10000
E = 320000
F = 128
HID = 128
C = 40

NC = 2
NS = 16
NW = NC * NS
EPW = E // NW
CH = 128
NCH = 80
KB = 8
NBLK = NCH // KB
TRASH = N
NH = 10240
HPT = NH // NS
NP = 10240
RPT = NP // NS

_mesh = plsc.VectorSubcoreMesh(core_axis_name="c", subcore_axis_name="s")


@functools.partial(
    pl.kernel,
    out_type=jax.ShapeDtypeStruct((NC, 2, NH), jnp.float32),
    mesh=_mesh,
    scratch_types=[
        pltpu.VMEM((2, NCH, CH), jnp.int32),
        pltpu.VMEM((CH,), jnp.float32),
        pltpu.VMEM((HPT,), jnp.float32),
        pltpu.VMEM_SHARED((NH,), jnp.float32),
        pltpu.VMEM_SHARED((NH,), jnp.float32),
    ],
)
def _degree_kernel(edge_hbm, out_hbm, idx_v, ones_v, zeros_v, hsrc_s, hdst_s):
    c = lax.axis_index("c")
    s = lax.axis_index("s")
    wid = s * NC + c
    pltpu.sync_copy(edge_hbm.at[wid], idx_v)
    one = jnp.ones((16,), jnp.float32)
    zero = jnp.zeros((16,), jnp.float32)
    for q in range(CH // 16):
        ones_v[pl.ds(q * 16, 16)] = one
    for q in range(HPT // 16):
        zeros_v[pl.ds(q * 16, 16)] = zero
    sl = pl.ds(s * HPT, HPT)
    pltpu.sync_copy(zeros_v, hsrc_s.at[sl])
    pltpu.sync_copy(zeros_v, hdst_s.at[sl])
    plsc.subcore_barrier()

    def body(i, carry):
        pltpu.sync_copy(ones_v, hsrc_s.at[idx_v.at[0, i]], add=True)
        pltpu.sync_copy(ones_v, hdst_s.at[idx_v.at[1, i]], add=True)
        return carry

    lax.fori_loop(0, NCH, body, 0)
    plsc.subcore_barrier()
    pltpu.sync_copy(hsrc_s.at[sl], out_hbm.at[c, 0, sl])
    pltpu.sync_copy(hdst_s.at[sl], out_hbm.at[c, 1, sl])


@functools.partial(
    pl.kernel,
    out_type=jax.ShapeDtypeStruct((NC, NP, HID), jnp.float32),
    mesh=_mesh,
    scratch_types=[
        pltpu.VMEM((2, 2, KB, CH), jnp.int32),
        pltpu.VMEM((2, CH, HID), jnp.float32),
        pltpu.VMEM_SHARED((NP, HID), jnp.float32),
        pltpu.SemaphoreType.DMA,
        pltpu.SemaphoreType.DMA,
    ],
)
def _agg_kernel(hn_hbm, edge_hbm, out_hbm, idx_v, rows_v, acc_s, sem0, sem1):
    c = lax.axis_index("c")
    s = lax.axis_index("s")
    wid = s * NC + c
    sems = (sem0, sem1)

    zero = jnp.zeros((16,), jnp.float32)
    for r in range(CH):
        for q in range(HID // 16):
            rows_v[0, r, pl.ds(q * 16, 16)] = zero
    for k in range(RPT // CH):
        pltpu.sync_copy(rows_v.at[0], acc_s.at[pl.ds(s * RPT + k * CH, CH)])
    plsc.subcore_barrier()

    pltpu.sync_copy(edge_hbm.at[wid, :, pl.ds(0, KB)], idx_v.at[0])
    pltpu.sync_copy(edge_hbm.at[wid, :, pl.ds(KB, KB)], idx_v.at[1])
    pltpu.async_copy(hn_hbm.at[idx_v.at[0, 0, 0]], rows_v.at[0], sems[0])
    pltpu.async_copy(hn_hbm.at[idx_v.at[0, 0, 1]], rows_v.at[1], sems[1])

    def block(k, carry):
        kb = lax.rem(k, 2)
        kb1 = lax.rem(k + 1, 2)
        for m in range(KB):
            b = m % 2
            pltpu.make_async_copy(
                hn_hbm.at[idx_v.at[kb, 0, m]], rows_v.at[b], sems[b]).wait()
            pltpu.sync_copy(rows_v.at[b], acc_s.at[idx_v.at[kb, 1, m]],
                            add=True)
            if m < KB - 2:
                pltpu.async_copy(
                    hn_hbm.at[idx_v.at[kb, 0, m + 2]], rows_v.at[b], sems[b])
            else:
                @pl.when(k < NBLK - 1)
                def _():
                    pltpu.async_copy(
                        hn_hbm.at[idx_v.at[kb1, 0, m + 2 - KB]],
                        rows_v.at[b], sems[b])
        @pl.when(k < NBLK - 2)
        def _():
            pltpu.sync_copy(
                edge_hbm.at[wid, :, pl.ds((k + 2) * KB, KB)], idx_v.at[kb])
        return carry

    lax.fori_loop(0, NBLK, block, 0)
    plsc.subcore_barrier()
    sl = pl.ds(s * RPT, RPT)
    pltpu.sync_copy(acc_s.at[sl], out_hbm.at[c, sl])


BM = 2000
GRID = N // BM


def _rsqrt_deg(h):
    return lax.rsqrt(jnp.maximum(h, 1.0))


def _proj_body(nf_ref, wp_ref, bp_ref, hp_ref, out_ref):
    ns = _rsqrt_deg(hp_ref[0, 0] + hp_ref[1, 0])
    x = jnp.dot(nf_ref[...], wp_ref[...],
                preferred_element_type=jnp.float32) + bp_ref[...]
    out_ref[...] = x * ns


def _conv_body(aggp_ref, hp_ref, w_ref, b_ref, out_ref):
    a = aggp_ref[0] + aggp_ref[1]
    nd = _rsqrt_deg(hp_ref[0, 1] + hp_ref[1, 1])
    ns = _rsqrt_deg(hp_ref[0, 0] + hp_ref[1, 0])
    h = jnp.maximum(
        jnp.dot(a * nd, w_ref[...], preferred_element_type=jnp.float32)
        + b_ref[...], 0.0)
    out_ref[...] = h * ns


def _head_body(aggp_ref, hp_ref, w2_ref, b2_ref, wc_ref, bc_ref, out_ref):
    a = aggp_ref[0] + aggp_ref[1]
    nd = _rsqrt_deg(hp_ref[0, 1] + hp_ref[1, 1])
    h = jnp.maximum(
        jnp.dot(a * nd, w2_ref[...], preferred_element_type=jnp.float32)
        + b2_ref[...], 0.0)
    out_ref[...] = jnp.dot(h, wc_ref[...],
                           preferred_element_type=jnp.float32) + bc_ref[...]


def _hp_spec():
    return pl.BlockSpec((NC, 2, BM, 1), lambda m: (0, 0, m, 0))


def _proj_call(n_feats, W_proj, b_proj2, hp):
    return pl.pallas_call(
        _proj_body,
        grid=(GRID,),
        in_specs=[
            pl.BlockSpec((BM, F), lambda m: (m, 0)),
            pl.BlockSpec((F, HID), lambda m: (0, 0)),
            pl.BlockSpec((1, HID), lambda m: (0, 0)),
            _hp_spec(),
        ],
        out_specs=pl.BlockSpec((BM, HID), lambda m: (m, 0)),
        out_shape=jax.ShapeDtypeStruct((N, HID), jnp.float32),
    )(n_feats, W_proj, b_proj2, hp)


def _conv_call(aggp, hp, W, b2):
    return pl.pallas_call(
        _conv_body,
        grid=(GRID,),
        in_specs=[
            pl.BlockSpec((NC, BM, HID), lambda m: (0, m, 0)),
            _hp_spec(),
            pl.BlockSpec((HID, HID), lambda m: (0, 0)),
            pl.BlockSpec((1, HID), lambda m: (0, 0)),
        ],
        out_specs=pl.BlockSpec((BM, HID), lambda m: (m, 0)),
        out_shape=jax.ShapeDtypeStruct((N, HID), jnp.float32),
    )(aggp, hp, W, b2)


def _head_call(aggp, hp, W2, b22, W_cls, b_cls2):
    return pl.pallas_call(
        _head_body,
        grid=(GRID,),
        in_specs=[
            pl.BlockSpec((NC, BM, HID), lambda m: (0, m, 0)),
            _hp_spec(),
            pl.BlockSpec((HID, HID), lambda m: (0, 0)),
            pl.BlockSpec((1, HID), lambda m: (0, 0)),
            pl.BlockSpec((HID, C), lambda m: (0, 0)),
            pl.BlockSpec((1, C), lambda m: (0, 0)),
        ],
        out_specs=pl.BlockSpec((BM, C), lambda m: (m, 0)),
        out_shape=jax.ShapeDtypeStruct((N, C), jnp.float32),
    )(aggp, hp, W2, b22, W_cls, b_cls2)


def kernel(edge_index, n_feats, W_proj, b_proj, W1, b1, W2, b2, W_cls, b_cls):
    ei = edge_index.reshape(2, NW, EPW)
    npad = NCH * CH - EPW
    pad_deg = jnp.full((2, NW, npad), TRASH, jnp.int32)
    pad_agg = jnp.stack([
        jnp.zeros((NW, npad), jnp.int32),
        jnp.full((NW, npad), TRASH, jnp.int32)])
    edge_deg = jnp.concatenate([ei, pad_deg], axis=2) \
        .reshape(2, NW, NCH, CH).transpose(1, 0, 2, 3)
    edge_agg = jnp.concatenate([ei, pad_agg], axis=2) \
        .reshape(2, NW, NCH, CH).transpose(1, 0, 2, 3)

    hp = _degree_kernel(edge_deg)
    hp4 = hp.reshape(NC, 2, NH, 1)
    hn1 = _proj_call(n_feats, W_proj, b_proj.reshape(1, HID), hp4)
    p1 = _agg_kernel(hn1, edge_agg)
    hn2 = _conv_call(p1, hp4, W1, b1.reshape(1, HID))
    p2 = _agg_kernel(hn2, edge_agg)
    out = _head_call(p2, hp4, W2, b2.reshape(1, HID), W_cls,
                     b_cls.reshape(1, C))
    return out
